# Initial kernel scaffold; baseline (speedup 1.0000x reference)
import jax
import jax.numpy as jnp
from jax import lax
from jax.experimental import pallas as pl
from jax.experimental.pallas import tpu as pltpu
from jax.experimental.pallas import tpu_sc as plsc

N_NODES = 50000
N_EDGES = 1600000

NC = 2
NS = 16
NW = NC * NS

EB = 128
ROWS = N_EDGES // EB
RPW = -(-ROWS // NW)

N_PAD = 50176
SPAN = N_PAD // NS


def _fill(rows, F, value):
    v = jnp.full((16,), value, jnp.float32)
    for j in range(EB * F // 16):
        if F == 1:
            rows[pl.ds(j * 16, 16)] = v
        else:
            r, q = divmod(j * 16, F)
            rows[r, pl.ds(q, 16)] = v


def _sc_phase(F, gather):
    vec = (N_PAD,) if F == 1 else (N_PAD, F)
    blk = (EB,) if F == 1 else (EB, F)
    out_t = (jax.ShapeDtypeStruct(vec, jnp.float32),
             jax.ShapeDtypeStruct(vec, jnp.float32))
    scratch = [
        pltpu.VMEM((EB,), jnp.int32),
        pltpu.VMEM((EB,), jnp.int32),
        pltpu.VMEM(blk, jnp.float32),
        pltpu.VMEM_SHARED(vec, jnp.float32),
    ]
    mesh = plsc.VectorSubcoreMesh(core_axis_name="c", subcore_axis_name="s",
                                  num_cores=NC, num_subcores=NS)

    def body(*refs):
        if gather:
            table, src2, dst2, out0, out1, idx_s, idx_d, rows, acc = refs
        else:
            src2, dst2, out0, out1, idx_s, idx_d, rows, acc = refs
            table = None
        c = lax.axis_index("c")
        s = lax.axis_index("s")
        w = s * NC + c

        _fill(rows, F, 0.0)
        nfull, rem = divmod(SPAN, EB)
        base = s * SPAN

        def zacc(i, carry):
            pltpu.sync_copy(rows, acc.at[pl.ds(base + i * EB, EB)])
            return carry

        lax.fori_loop(0, nfull, zacc, 0)
        if rem:
            pltpu.sync_copy(rows.at[pl.ds(0, rem)],
                            acc.at[pl.ds(base + nfull * EB, rem)])

        if not gather:
            _fill(rows, F, 1.0)

        plsc.subcore_barrier()

        start = w * RPW
        count = jnp.clip(ROWS - start, 0, RPW)

        def ebody(i, carry):
            r = start + i
            pltpu.sync_copy(src2.at[r], idx_s)
            pltpu.sync_copy(dst2.at[r], idx_d)
            if gather:
                pltpu.sync_copy(table.at[idx_s], rows)
            pltpu.sync_copy(rows, acc.at[idx_d], add=True)
            return carry

        lax.fori_loop(0, count, ebody, 0)

        plsc.subcore_barrier()

        sl = pl.ds(base, SPAN)

        @pl.when(c == 0)
        def _():
            pltpu.sync_copy(acc.at[sl], out0.at[sl])

        @pl.when(c == 1)
        def _():
            pltpu.sync_copy(acc.at[sl], out1.at[sl])

    return pl.kernel(body, out_type=out_t, mesh=mesh, scratch_types=scratch)


BLK = 784
GRID = N_PAD // BLK


def _vspec():
    return pl.BlockSpec((BLK, 1), lambda i: (i, 0))


def _fspec(F):
    return pl.BlockSpec((BLK, F), lambda i: (i, 0))


def _wspec(shape):
    nd = len(shape)
    return pl.BlockSpec(shape, lambda i: (0,) * nd)


def _tc_call(fn, in_specs, out_specs, out_shapes):
    return pl.pallas_call(
        fn,
        grid=(GRID,),
        in_specs=in_specs,
        out_specs=out_specs,
        out_shape=out_shapes,
    )


def _tc_a_body(degp0, degp1, x, dinv_o, xp1_o):
    deg = degp0[...] + degp1[...] + 1.0
    dinv = lax.rsqrt(deg)
    dinv_o[...] = dinv
    xp1_o[...] = dinv * x[...]


def _tc_b_body(p0, p1, xp1, dinv, w1, b1, w2, xp2_o):
    p = dinv[...] * (p0[...] + p1[...] + xp1[...])
    h1 = jnp.maximum(p * w1[...] + b1[...][None, :], 0.0)
    g1 = jnp.dot(h1, w2[...], preferred_element_type=jnp.float32)
    xp2_o[...] = dinv[...] * g1


def _tc_mid_body(p0, p1, xp, dinv, b, w, xp_next_o):
    h = jnp.maximum(
        dinv[...] * (p0[...] + p1[...] + xp[...]) + b[...][None, :], 0.0)
    g = jnp.dot(h, w[...], preferred_element_type=jnp.float32)
    xp_next_o[...] = dinv[...] * g


def _tc_e_body(p0, p1, xp4, dinv, b4, out_o):
    out_o[...] = dinv[...] * (p0[...] + p1[...] + xp4[...]) + b4[...][None, :]


def kernel(x, edge_index, W1, b1, W2, b2, W3, b3, W4, b4):
    f32 = jnp.float32
    src2 = edge_index[0].reshape(ROWS, EB)
    dst2 = edge_index[1].reshape(ROWS, EB)
    xpad = jnp.pad(x, ((0, N_PAD - N_NODES), (0, 0)))

    sc_deg = _sc_phase(1, gather=False)
    sc_f1 = _sc_phase(1, gather=True)
    sc_f32 = _sc_phase(32, gather=True)
    sc_f16 = _sc_phase(16, gather=True)

    d0, d1 = sc_deg(src2, dst2)

    v = jax.ShapeDtypeStruct((N_PAD, 1), f32)
    tc_a = _tc_call(_tc_a_body, [_vspec()] * 3, [_vspec()] * 2, [v, v])
    dinv, xp1 = tc_a(d0[:, None], d1[:, None], xpad)

    q0, q1 = sc_f1(xp1[:, 0], src2, dst2)

    tc_b = _tc_call(
        _tc_b_body,
        [_vspec()] * 4 + [_wspec((1, 64)), _wspec((64,)), _wspec((64, 32))],
        [_fspec(32)],
        [jax.ShapeDtypeStruct((N_PAD, 32), f32)],
    )
    (xp2,) = tc_b(q0[:, None], q1[:, None], xp1, dinv, W1, b1, W2)

    r0, r1 = sc_f32(xp2, src2, dst2)

    tc_c = _tc_call(
        _tc_mid_body,
        [_fspec(32)] * 3 + [_vspec(), _wspec((32,)), _wspec((32, 16))],
        [_fspec(16)],
        [jax.ShapeDtypeStruct((N_PAD, 16), f32)],
    )
    (xp3,) = tc_c(r0, r1, xp2, dinv, b2, W3)

    s0, s1 = sc_f16(xp3, src2, dst2)

    tc_d = _tc_call(
        _tc_mid_body,
        [_fspec(16)] * 3 + [_vspec(), _wspec((16,)), _wspec((16, 1))],
        [_vspec()],
        [v],
    )
    (xp4,) = tc_d(s0, s1, xp3, dinv, b3, W4)

    t0, t1 = sc_f1(xp4[:, 0], src2, dst2)

    tc_e = _tc_call(
        _tc_e_body,
        [_vspec()] * 4 + [_wspec((1,))],
        [_vspec()],
        [v],
    )
    (out,) = tc_e(t0[:, None], t1[:, None], xp4, dinv, b4)
    return out[:N_NODES]


# speedup vs baseline: 16.0750x; 16.0750x over previous
import jax
import jax.numpy as jnp
from jax import lax
from jax.experimental import pallas as pl
from jax.experimental.pallas import tpu as pltpu
from jax.experimental.pallas import tpu_sc as plsc

N_NODES = 50000
N_EDGES = 1600000

NC = 2
NS = 16
NW = NC * NS

EB = 128
ROWS = N_EDGES // EB
RPW = -(-ROWS // NW)

N_PAD = 50176
SPAN = N_PAD // NS


def _fill(rows, F, value):
    v = jnp.full((16,), value, jnp.float32)
    for j in range(EB * F // 16):
        if F == 1:
            rows[pl.ds(j * 16, 16)] = v
        else:
            r, q = divmod(j * 16, F)
            rows[r, pl.ds(q, 16)] = v


def _sc_phase(F, gather):
    vec = (N_PAD,) if F == 1 else (N_PAD, F)
    blk = (EB,) if F == 1 else (EB, F)
    out_t = (jax.ShapeDtypeStruct(vec, jnp.float32),
             jax.ShapeDtypeStruct(vec, jnp.float32))
    bshape = (SPAN,) if F == 1 else (SPAN, F)
    scratch = [
        pltpu.VMEM((EB,), jnp.int32),
        pltpu.VMEM((EB,), jnp.int32),
        pltpu.VMEM(blk, jnp.float32),
        pltpu.VMEM(bshape, jnp.float32),
        pltpu.VMEM_SHARED(vec, jnp.float32),
    ]
    mesh = plsc.VectorSubcoreMesh(core_axis_name="c", subcore_axis_name="s",
                                  num_cores=NC, num_subcores=NS)

    def body(*refs):
        if gather:
            table, src2, dst2, out0, out1, idx_s, idx_d, rows, bounce, acc = refs
        else:
            src2, dst2, out0, out1, idx_s, idx_d, rows, bounce, acc = refs
            table = None
        c = lax.axis_index("c")
        s = lax.axis_index("s")
        w = s * NC + c

        _fill(rows, F, 0.0)
        nfull, rem = divmod(SPAN, EB)
        base = s * SPAN

        def zacc(i, carry):
            pltpu.sync_copy(rows, acc.at[pl.ds(base + i * EB, EB)])
            return carry

        lax.fori_loop(0, nfull, zacc, 0)
        if rem:
            pltpu.sync_copy(rows.at[pl.ds(0, rem)],
                            acc.at[pl.ds(base + nfull * EB, rem)])

        if not gather:
            _fill(rows, F, 1.0)

        plsc.subcore_barrier()

        start = w * RPW
        count = jnp.clip(ROWS - start, 0, RPW)

        def ebody(i, carry):
            r = start + i
            pltpu.sync_copy(src2.at[r], idx_s)
            pltpu.sync_copy(dst2.at[r], idx_d)
            if gather:
                pltpu.sync_copy(table.at[idx_s], rows)
            pltpu.sync_copy(rows, acc.at[idx_d], add=True)
            return carry

        lax.fori_loop(0, count, ebody, 0)

        plsc.subcore_barrier()

        sl = pl.ds(base, SPAN)
        pltpu.sync_copy(acc.at[sl], bounce)

        @pl.when(c == 0)
        def _():
            pltpu.sync_copy(bounce, out0.at[sl])

        @pl.when(c == 1)
        def _():
            pltpu.sync_copy(bounce, out1.at[sl])

    return pl.kernel(
        body, out_type=out_t, mesh=mesh, scratch_types=scratch,
        compiler_params=pltpu.CompilerParams(use_tc_tiling_on_sc=False))


BLK = 784
GRID = N_PAD // BLK


def _vspec():
    return pl.BlockSpec((BLK, 1), lambda i: (i, 0))


def _fspec(F):
    return pl.BlockSpec((BLK, F), lambda i: (i, 0))


def _wspec(shape):
    nd = len(shape)
    return pl.BlockSpec(shape, lambda i: (0,) * nd)


def _tc_call(fn, in_specs, out_specs, out_shapes):
    return pl.pallas_call(
        fn,
        grid=(GRID,),
        in_specs=in_specs,
        out_specs=out_specs,
        out_shape=out_shapes,
    )


def _tc_a_body(degp0, degp1, x, dinv_o, xp1_o):
    deg = degp0[...] + degp1[...] + 1.0
    dinv = lax.rsqrt(deg)
    dinv_o[...] = dinv
    xp1_o[...] = dinv * x[...]


def _tc_b_body(p0, p1, xp1, dinv, w1, b1, w2, xp2_o):
    p = dinv[...] * (p0[...] + p1[...] + xp1[...])
    h1 = jnp.maximum(p * w1[...] + b1[...][None, :], 0.0)
    g1 = jnp.dot(h1, w2[...], preferred_element_type=jnp.float32)
    xp2_o[...] = dinv[...] * g1


def _tc_c_body(p0a, p1a, p0b, p1b, xp2, dinv, b, w, xp3_o):
    agg = jnp.concatenate([p0a[...] + p1a[...], p0b[...] + p1b[...]], axis=1)
    h = jnp.maximum(dinv[...] * (agg + xp2[...]) + b[...][None, :], 0.0)
    g = jnp.dot(h, w[...], preferred_element_type=jnp.float32)
    xp3_o[...] = dinv[...] * g


def _tc_d_body(p0, p1, xp, dinv, b, w, xp_next_o):
    h = jnp.maximum(
        dinv[...] * (p0[...] + p1[...] + xp[...]) + b[...][None, :], 0.0)
    g = jnp.dot(h, w[...], preferred_element_type=jnp.float32)
    xp_next_o[...] = dinv[...] * g


def _tc_e_body(p0, p1, xp4, dinv, b4, out_o):
    out_o[...] = dinv[...] * (p0[...] + p1[...] + xp4[...]) + b4[...][None, :]


def kernel(x, edge_index, W1, b1, W2, b2, W3, b3, W4, b4):
    f32 = jnp.float32
    src2 = edge_index[0].reshape(ROWS, EB)
    dst2 = edge_index[1].reshape(ROWS, EB)
    xpad = jnp.pad(x, ((0, N_PAD - N_NODES), (0, 0)))

    sc_deg = _sc_phase(1, gather=False)
    sc_f1 = _sc_phase(1, gather=True)
    sc_f16 = _sc_phase(16, gather=True)

    d0, d1 = sc_deg(src2, dst2)

    v = jax.ShapeDtypeStruct((N_PAD, 1), f32)
    tc_a = _tc_call(_tc_a_body, [_vspec()] * 3, [_vspec()] * 2, [v, v])
    dinv, xp1 = tc_a(d0[:, None], d1[:, None], xpad)

    q0, q1 = sc_f1(xp1[:, 0], src2, dst2)

    tc_b = _tc_call(
        _tc_b_body,
        [_vspec()] * 4 + [_wspec((1, 64)), _wspec((64,)), _wspec((64, 32))],
        [_fspec(32)],
        [jax.ShapeDtypeStruct((N_PAD, 32), f32)],
    )
    (xp2,) = tc_b(q0[:, None], q1[:, None], xp1, dinv, W1, b1, W2)

    r0a, r1a = sc_f16(xp2[:, :16], src2, dst2)
    r0b, r1b = sc_f16(xp2[:, 16:], src2, dst2)

    tc_c = _tc_call(
        _tc_c_body,
        [_fspec(16)] * 4 + [_fspec(32), _vspec(), _wspec((32,)),
                            _wspec((32, 16))],
        [_fspec(16)],
        [jax.ShapeDtypeStruct((N_PAD, 16), f32)],
    )
    (xp3,) = tc_c(r0a, r1a, r0b, r1b, xp2, dinv, b2, W3)

    s0, s1 = sc_f16(xp3, src2, dst2)

    tc_d = _tc_call(
        _tc_d_body,
        [_fspec(16)] * 3 + [_vspec(), _wspec((16,)), _wspec((16, 1))],
        [_vspec()],
        [v],
    )
    (xp4,) = tc_d(s0, s1, xp3, dinv, b3, W4)

    t0, t1 = sc_f1(xp4[:, 0], src2, dst2)

    tc_e = _tc_call(
        _tc_e_body,
        [_vspec()] * 4 + [_wspec((1,))],
        [_vspec()],
        [v],
    )
    (out,) = tc_e(t0[:, None], t1[:, None], xp4, dinv, b4)
    return out[:N_NODES]
